# baseline (speedup 1.0000x reference)
import functools

import jax
import jax.numpy as jnp
from jax import lax
from jax.experimental import pallas as pl
from jax.experimental.pallas import tpu as pltpu
from jax.experimental.pallas import tpu_sc as plsc

NC = 2
NS = 16
NW = NC * NS
K = 128
CPW = 80
EPAD = NW * CPW * K
TCH = NW * CPW
CPP = TCH // NS
X0 = 112
X1 = CPP - X0
CH_PAD = (NS - 1) * CPP + 2 * X0
NACC = 10112
RPTA = NACC // NS
DW = 16
DH = 64
ZR = 79

_mesh = functools.partial(
    plsc.VectorSubcoreMesh,
    core_axis_name="c", subcore_axis_name="s", num_cores=NC, num_subcores=NS,
)
_sc_params = pltpu.CompilerParams(use_tc_tiling_on_sc=False)


def _deg_body(dst_hbm, zeros_hbm, out_hbm, dst_v, ones_v, acc):
    c = lax.axis_index("c")
    s = lax.axis_index("s")
    wid = s * NC + c
    pltpu.sync_copy(dst_hbm.at[wid], dst_v)
    for i in range(K):
        ones_v[i] = jnp.ones((DW,), jnp.float32)
    pltpu.sync_copy(zeros_hbm.at[pl.ds(s * RPTA, RPTA)],
                    acc.at[pl.ds(s * RPTA, RPTA)])
    plsc.subcore_barrier()

    def chunk(j, _):
        pltpu.sync_copy(ones_v, acc.at[dst_v.at[j]], add=True)
        return ()

    lax.fori_loop(0, CPW, chunk, ())
    plsc.subcore_barrier()
    pltpu.sync_copy(acc.at[pl.ds(s * RPTA, RPTA)],
                    out_hbm.at[c, pl.ds(s * RPTA, RPTA)])


def _degree(dst3, zeros_deg):
    return pl.kernel(
        _deg_body,
        out_type=jax.ShapeDtypeStruct((NC, NACC, DW), jnp.float32),
        mesh=_mesh(),
        compiler_params=_sc_params,
        scratch_types=[
            pltpu.VMEM((CPW, K), jnp.int32),
            pltpu.VMEM((K, DW), jnp.float32),
            pltpu.VMEM_SHARED((NACC, DW), jnp.float32),
        ],
    )(dst3, zeros_deg)


def _fill_zeros(zbuf):
    for i in range(ZR):
        for k in range(DH // 16):
            zbuf[i, pl.ds(k * 16, 16)] = jnp.zeros((16,), jnp.float32)


def _zero_acc_local(zbuf, acc, s):
    for k in range(RPTA // ZR):
        pltpu.sync_copy(zbuf, acc.at[pl.ds(s * RPTA + k * ZR, ZR)])


NBUF = 4


def _seg_pass(h_hbm, src_v, dst_v, acc, bufs, sems, cnt):
    for b in range(NBUF):
        pltpu.async_copy(h_hbm.at[src_v.at[b]], bufs[b], sems[b])

    def step(i, _):
        g = i * NBUF
        for b in range(NBUF):
            j = g + b
            pltpu.make_async_copy(h_hbm.at[src_v.at[j]], bufs[b], sems[b]).wait()
            pltpu.sync_copy(bufs[b], acc.at[dst_v.at[j]], add=True)

            @pl.when(j + NBUF < cnt)
            def _():
                pltpu.async_copy(h_hbm.at[src_v.at[j + NBUF]], bufs[b], sems[b])
        return ()

    lax.fori_loop(0, cnt // NBUF, step, ())


def _seg_split(c, s):
    base = s * CPP + c * X0
    cnt = jnp.where(c == 0, X0, X1)
    return base, cnt


def _seg1_body(src_hbm, dst_hbm, hb_hbm, out_hbm,
               src_v, dst_v, buf0, buf1, buf2, buf3, zbuf, acc,
               sem0, sem1, sem2, sem3):
    c = lax.axis_index("c")
    s = lax.axis_index("s")
    base, cnt = _seg_split(c, s)
    pltpu.sync_copy(src_hbm.at[pl.ds(base, X0)], src_v)
    pltpu.sync_copy(dst_hbm.at[pl.ds(base, X0)], dst_v)
    for i in range(ZR):
        for k in range(128 // 32):
            zbuf[i, pl.ds(k * 32, 32)] = jnp.zeros((32,), jnp.bfloat16)
    for k in range(RPTA // ZR):
        pltpu.sync_copy(zbuf, acc.at[pl.ds(s * RPTA + k * ZR, ZR)])
    plsc.subcore_barrier()
    _seg_pass(hb_hbm, src_v, dst_v, acc, (buf0, buf1, buf2, buf3),
              (sem0, sem1, sem2, sem3), cnt)
    plsc.subcore_barrier()
    pltpu.sync_copy(acc.at[pl.ds(s * RPTA, RPTA)],
                    out_hbm.at[c, pl.ds(s * RPTA, RPTA)])


def _segment_sum2(src2p, dst2p, hb):
    d_h = hb.shape[1]
    return pl.kernel(
        _seg1_body,
        out_type=jax.ShapeDtypeStruct((NC, NACC, d_h), jnp.bfloat16),
        mesh=_mesh(),
        compiler_params=_sc_params,
        scratch_types=[
            pltpu.VMEM((X0, K), jnp.int32),
            pltpu.VMEM((X0, K), jnp.int32),
            pltpu.VMEM((K, d_h), jnp.bfloat16),
            pltpu.VMEM((K, d_h), jnp.bfloat16),
            pltpu.VMEM((K, d_h), jnp.bfloat16),
            pltpu.VMEM((K, d_h), jnp.bfloat16),
            pltpu.VMEM((ZR, d_h), jnp.bfloat16),
            pltpu.VMEM_SHARED((NACC, d_h), jnp.bfloat16),
            pltpu.SemaphoreType.DMA,
            pltpu.SemaphoreType.DMA,
            pltpu.SemaphoreType.DMA,
            pltpu.SemaphoreType.DMA,
        ],
    )(src2p, dst2p, hb)


def _seg2_body(src_hbm, dst_hbm, h_hbm, out_hbm,
               src_v, dst_v, buf0, buf1, buf2, buf3, zbuf, acc,
               sem0, sem1, sem2, sem3):
    c = lax.axis_index("c")
    s = lax.axis_index("s")
    base, cnt = _seg_split(c, s)
    pltpu.sync_copy(src_hbm.at[pl.ds(base, X0)], src_v)
    pltpu.sync_copy(dst_hbm.at[pl.ds(base, X0)], dst_v)
    _fill_zeros(zbuf)
    _zero_acc_local(zbuf, acc, s)
    plsc.subcore_barrier()
    _seg_pass(h_hbm, src_v, dst_v, acc, (buf0, buf1, buf2, buf3),
              (sem0, sem1, sem2, sem3), cnt)
    plsc.subcore_barrier()
    pltpu.sync_copy(acc.at[pl.ds(s * RPTA, RPTA)],
                    out_hbm.at[c, pl.ds(s * RPTA, RPTA)])


def _segment_sum(src2p, dst2p, h):
    return pl.kernel(
        _seg2_body,
        out_type=jax.ShapeDtypeStruct((NC, NACC, DH), jnp.float32),
        mesh=_mesh(),
        compiler_params=_sc_params,
        scratch_types=[
            pltpu.VMEM((X0, K), jnp.int32),
            pltpu.VMEM((X0, K), jnp.int32),
            pltpu.VMEM((K, DH), jnp.float32),
            pltpu.VMEM((K, DH), jnp.float32),
            pltpu.VMEM((K, DH), jnp.float32),
            pltpu.VMEM((K, DH), jnp.float32),
            pltpu.VMEM((ZR, DH), jnp.float32),
            pltpu.VMEM_SHARED((NACC, DH), jnp.float32),
            pltpu.SemaphoreType.DMA,
            pltpu.SemaphoreType.DMA,
            pltpu.SemaphoreType.DMA,
            pltpu.SemaphoreType.DMA,
        ],
    )(src2p, dst2p, h)


_BR = 632
_GRID = NACC // _BR


def _mm_scale_body(x_ref, w_ref, dinv_ref, o_ref):
    h = jnp.dot(x_ref[...], w_ref[...], preferred_element_type=jnp.float32)
    o_ref[...] = (h * dinv_ref[...]).astype(jnp.bfloat16)


def _mm_scale(xp, w, dinv):
    d_in, d_out = w.shape
    return pl.pallas_call(
        _mm_scale_body,
        grid=(_GRID,),
        in_specs=[
            pl.BlockSpec((_BR, d_in), lambda i: (i, 0)),
            pl.BlockSpec((d_in, d_out), lambda i: (0, 0)),
            pl.BlockSpec((_BR, 1), lambda i: (i, 0)),
        ],
        out_specs=pl.BlockSpec((_BR, d_out), lambda i: (i, 0)),
        out_shape=jax.ShapeDtypeStruct((NACC, d_out), jnp.bfloat16),
    )(xp, w, dinv)


def _mid_body(acc_ref, hb_ref, dinv_ref, b1_ref, w2_ref, o_ref):
    a = (acc_ref[0].astype(jnp.float32) + acc_ref[1].astype(jnp.float32)
         + hb_ref[...].astype(jnp.float32))
    z = a * dinv_ref[...] + b1_ref[...]
    z = jnp.maximum(z, 0.0)
    h2 = jnp.dot(z, w2_ref[...], preferred_element_type=jnp.float32)
    o_ref[...] = h2 * dinv_ref[...]


def _mid(acc1, hb, dinv, b1, w2):
    d_in, d_out = w2.shape
    return pl.pallas_call(
        _mid_body,
        grid=(_GRID,),
        in_specs=[
            pl.BlockSpec((NC, _BR, d_in), lambda i: (0, i, 0)),
            pl.BlockSpec((_BR, d_in), lambda i: (i, 0)),
            pl.BlockSpec((_BR, 1), lambda i: (i, 0)),
            pl.BlockSpec((1, d_in), lambda i: (0, 0)),
            pl.BlockSpec((d_in, d_out), lambda i: (0, 0)),
        ],
        out_specs=pl.BlockSpec((_BR, d_out), lambda i: (i, 0)),
        out_shape=jax.ShapeDtypeStruct((NACC, d_out), jnp.float32),
    )(acc1, hb, dinv, b1, w2)


def _post_body(acc_ref, hp_ref, dinv_ref, b2_ref, o_ref):
    t = (acc_ref[0] + acc_ref[1] + hp_ref[...]) * dinv_ref[...] + b2_ref[...]
    m = jnp.max(t, axis=1, keepdims=True)
    e = jnp.exp(t - m)
    lse = jnp.log(jnp.sum(e, axis=1, keepdims=True)) + m
    o_ref[...] = t - lse


def _post(acc2, h2p, dinv, b2):
    d = h2p.shape[1]
    return pl.pallas_call(
        _post_body,
        grid=(_GRID,),
        in_specs=[
            pl.BlockSpec((NC, _BR, d), lambda i: (0, i, 0)),
            pl.BlockSpec((_BR, d), lambda i: (i, 0)),
            pl.BlockSpec((_BR, 1), lambda i: (i, 0)),
            pl.BlockSpec((1, d), lambda i: (0, 0)),
        ],
        out_specs=pl.BlockSpec((_BR, d), lambda i: (i, 0)),
        out_shape=jax.ShapeDtypeStruct((NACC, d), jnp.float32),
    )(acc2, h2p, dinv, b2)


def kernel(x, edge_index, W1, b1, W2, b2):
    n, d_in = x.shape
    e = edge_index.shape[1]
    d_h = W1.shape[1]
    d_out = W2.shape[1]

    xp = jnp.pad(x, ((0, NACC - n), (0, 0)))
    pad_e = EPAD - e
    src1 = jnp.concatenate([edge_index[0], jnp.zeros((pad_e,), jnp.int32)])
    dst1 = jnp.concatenate([edge_index[1], jnp.full((pad_e,), n, jnp.int32)])
    src2 = src1.reshape(TCH, K)
    dst2 = dst1.reshape(TCH, K)
    pad_ch = CH_PAD - TCH
    src2p = jnp.concatenate([src2, jnp.zeros((pad_ch, K), jnp.int32)])
    dst2p = jnp.concatenate([dst2, jnp.full((pad_ch, K), n, jnp.int32)])
    dst3 = dst2.reshape(NW, CPW, K)
    zeros_deg = jnp.zeros((NACC, DW), jnp.float32)

    deg_parts = _degree(dst3, zeros_deg)
    deg = deg_parts[0, :, 0] + deg_parts[1, :, 0]
    dinv = lax.rsqrt(deg + 1.0).reshape(NACC, 1)

    hb = _mm_scale(xp, W1, dinv)
    acc1 = _segment_sum2(src2p, dst2p, hb)
    h2p = _mid(acc1, hb, dinv, b1.reshape(1, d_h), W2)
    acc2 = _segment_sum(src2p, dst2p, h2p)
    out = _post(acc2, h2p, dinv, b2.reshape(1, d_out))
    return out[:n]


# speedup vs baseline: 1.2319x; 1.2319x over previous
import functools

import jax
import jax.numpy as jnp
from jax import lax
from jax.experimental import pallas as pl
from jax.experimental.pallas import tpu as pltpu
from jax.experimental.pallas import tpu_sc as plsc

NC = 2
NS = 16
NW = NC * NS
K = 128
CPW = 80
EPAD = NW * CPW * K
TCH = NW * CPW
CPP = TCH // NS
X0 = 112
X1 = CPP - X0
CH_PAD = (NS - 1) * CPP + 2 * X0
NACC = 10112
RPTA = NACC // NS
DW = 16
DH = 64
ZR = 79

_mesh = functools.partial(
    plsc.VectorSubcoreMesh,
    core_axis_name="c", subcore_axis_name="s", num_cores=NC, num_subcores=NS,
)
_sc_params = pltpu.CompilerParams(use_tc_tiling_on_sc=False)


def _deg_body(dst_hbm, zeros_hbm, out_hbm, dst_v, ones_v, acc):
    c = lax.axis_index("c")
    s = lax.axis_index("s")
    wid = s * NC + c
    pltpu.sync_copy(dst_hbm.at[wid], dst_v)
    for i in range(K):
        ones_v[i] = jnp.ones((DW,), jnp.float32)
    pltpu.sync_copy(zeros_hbm.at[pl.ds(s * RPTA, RPTA)],
                    acc.at[pl.ds(s * RPTA, RPTA)])
    plsc.subcore_barrier()

    def chunk(j, _):
        pltpu.sync_copy(ones_v, acc.at[dst_v.at[j]], add=True)
        return ()

    lax.fori_loop(0, CPW, chunk, ())
    plsc.subcore_barrier()
    pltpu.sync_copy(acc.at[pl.ds(s * RPTA, RPTA)],
                    out_hbm.at[c, pl.ds(s * RPTA, RPTA)])


def _degree(dst3, zeros_deg):
    return pl.kernel(
        _deg_body,
        out_type=jax.ShapeDtypeStruct((NC, NACC, DW), jnp.float32),
        mesh=_mesh(),
        compiler_params=_sc_params,
        scratch_types=[
            pltpu.VMEM((CPW, K), jnp.int32),
            pltpu.VMEM((K, DW), jnp.float32),
            pltpu.VMEM_SHARED((NACC, DW), jnp.float32),
        ],
    )(dst3, zeros_deg)


def _fill_zeros(zbuf):
    for i in range(ZR):
        for k in range(DH // 16):
            zbuf[i, pl.ds(k * 16, 16)] = jnp.zeros((16,), jnp.float32)


def _zero_acc_local(zbuf, acc, s):
    for k in range(RPTA // ZR):
        pltpu.sync_copy(zbuf, acc.at[pl.ds(s * RPTA + k * ZR, ZR)])


NBUF = 2


def _seg_pass(h_hbm, src_v, dst_v, acc, bufs, sems, cnt):
    for b in range(NBUF):
        pltpu.async_copy(h_hbm.at[src_v.at[b]], bufs[b], sems[b])

    def step(i, _):
        g = i * NBUF
        for b in range(NBUF):
            j = g + b
            pltpu.make_async_copy(h_hbm.at[src_v.at[j]], bufs[b], sems[b]).wait()
            pltpu.sync_copy(bufs[b], acc.at[dst_v.at[j]], add=True)

            @pl.when(j + NBUF < cnt)
            def _():
                pltpu.async_copy(h_hbm.at[src_v.at[j + NBUF]], bufs[b], sems[b])
        return ()

    lax.fori_loop(0, cnt // NBUF, step, ())


def _seg_split(c, s):
    base = s * CPP + c * X0
    cnt = jnp.where(c == 0, X0, X1)
    return base, cnt


def _seg1_body(src_hbm, dst_hbm, hb_hbm, out_hbm,
               src_v, dst_v, buf0, buf1, zbuf, acc, sem0, sem1):
    c = lax.axis_index("c")
    s = lax.axis_index("s")
    base, cnt = _seg_split(c, s)
    pltpu.sync_copy(src_hbm.at[pl.ds(base, X0)], src_v)
    pltpu.sync_copy(dst_hbm.at[pl.ds(base, X0)], dst_v)
    for i in range(ZR):
        for k in range(128 // 32):
            zbuf[i, pl.ds(k * 32, 32)] = jnp.zeros((32,), jnp.bfloat16)
    for k in range(RPTA // ZR):
        pltpu.sync_copy(zbuf, acc.at[pl.ds(s * RPTA + k * ZR, ZR)])
    plsc.subcore_barrier()
    _seg_pass(hb_hbm, src_v, dst_v, acc, (buf0, buf1), (sem0, sem1), cnt)
    plsc.subcore_barrier()
    pltpu.sync_copy(acc.at[pl.ds(s * RPTA, RPTA)],
                    out_hbm.at[c, pl.ds(s * RPTA, RPTA)])


def _segment_sum2(src2p, dst2p, hb):
    d_h = hb.shape[1]
    return pl.kernel(
        _seg1_body,
        out_type=jax.ShapeDtypeStruct((NC, NACC, d_h), jnp.bfloat16),
        mesh=_mesh(),
        compiler_params=_sc_params,
        scratch_types=[
            pltpu.VMEM((X0, K), jnp.int32),
            pltpu.VMEM((X0, K), jnp.int32),
            pltpu.VMEM((K, d_h), jnp.bfloat16),
            pltpu.VMEM((K, d_h), jnp.bfloat16),
            pltpu.VMEM((ZR, d_h), jnp.bfloat16),
            pltpu.VMEM_SHARED((NACC, d_h), jnp.bfloat16),
            pltpu.SemaphoreType.DMA,
            pltpu.SemaphoreType.DMA,
        ],
    )(src2p, dst2p, hb)


def _seg2_body(src_hbm, dst_hbm, h_hbm, out_hbm,
               src_v, dst_v, buf0, buf1, zbuf, acc, sem0, sem1):
    c = lax.axis_index("c")
    s = lax.axis_index("s")
    base, cnt = _seg_split(c, s)
    pltpu.sync_copy(src_hbm.at[pl.ds(base, X0)], src_v)
    pltpu.sync_copy(dst_hbm.at[pl.ds(base, X0)], dst_v)
    for i in range(ZR):
        for k in range(DH // 32):
            zbuf[i, pl.ds(k * 32, 32)] = jnp.zeros((32,), jnp.bfloat16)
    _zero_acc_local(zbuf, acc, s)
    plsc.subcore_barrier()
    _seg_pass(h_hbm, src_v, dst_v, acc, (buf0, buf1), (sem0, sem1), cnt)
    plsc.subcore_barrier()
    pltpu.sync_copy(acc.at[pl.ds(s * RPTA, RPTA)],
                    out_hbm.at[c, pl.ds(s * RPTA, RPTA)])


def _segment_sum(src2p, dst2p, h):
    return pl.kernel(
        _seg2_body,
        out_type=jax.ShapeDtypeStruct((NC, NACC, DH), jnp.bfloat16),
        mesh=_mesh(),
        compiler_params=_sc_params,
        scratch_types=[
            pltpu.VMEM((X0, K), jnp.int32),
            pltpu.VMEM((X0, K), jnp.int32),
            pltpu.VMEM((K, DH), jnp.bfloat16),
            pltpu.VMEM((K, DH), jnp.bfloat16),
            pltpu.VMEM((ZR, DH), jnp.bfloat16),
            pltpu.VMEM_SHARED((NACC, DH), jnp.bfloat16),
            pltpu.SemaphoreType.DMA,
            pltpu.SemaphoreType.DMA,
        ],
    )(src2p, dst2p, h)


_BR = 632
_GRID = NACC // _BR


def _mm_scale_body(x_ref, w_ref, dinv_ref, o_ref):
    h = jnp.dot(x_ref[...], w_ref[...], preferred_element_type=jnp.float32)
    o_ref[...] = (h * dinv_ref[...]).astype(jnp.bfloat16)


def _mm_scale(xp, w, dinv):
    d_in, d_out = w.shape
    return pl.pallas_call(
        _mm_scale_body,
        grid=(_GRID,),
        in_specs=[
            pl.BlockSpec((_BR, d_in), lambda i: (i, 0)),
            pl.BlockSpec((d_in, d_out), lambda i: (0, 0)),
            pl.BlockSpec((_BR, 1), lambda i: (i, 0)),
        ],
        out_specs=pl.BlockSpec((_BR, d_out), lambda i: (i, 0)),
        out_shape=jax.ShapeDtypeStruct((NACC, d_out), jnp.bfloat16),
    )(xp, w, dinv)


def _mid_body(acc_ref, hb_ref, dinv_ref, b1_ref, w2_ref, o_ref):
    a = (acc_ref[0].astype(jnp.float32) + acc_ref[1].astype(jnp.float32)
         + hb_ref[...].astype(jnp.float32))
    z = a * dinv_ref[...] + b1_ref[...]
    z = jnp.maximum(z, 0.0)
    h2 = jnp.dot(z, w2_ref[...], preferred_element_type=jnp.float32)
    o_ref[...] = (h2 * dinv_ref[...]).astype(jnp.bfloat16)


def _mid(acc1, hb, dinv, b1, w2):
    d_in, d_out = w2.shape
    return pl.pallas_call(
        _mid_body,
        grid=(_GRID,),
        in_specs=[
            pl.BlockSpec((NC, _BR, d_in), lambda i: (0, i, 0)),
            pl.BlockSpec((_BR, d_in), lambda i: (i, 0)),
            pl.BlockSpec((_BR, 1), lambda i: (i, 0)),
            pl.BlockSpec((1, d_in), lambda i: (0, 0)),
            pl.BlockSpec((d_in, d_out), lambda i: (0, 0)),
        ],
        out_specs=pl.BlockSpec((_BR, d_out), lambda i: (i, 0)),
        out_shape=jax.ShapeDtypeStruct((NACC, d_out), jnp.bfloat16),
    )(acc1, hb, dinv, b1, w2)


def _post_body(acc_ref, hp_ref, dinv_ref, b2_ref, o_ref):
    a = (acc_ref[0].astype(jnp.float32) + acc_ref[1].astype(jnp.float32)
         + hp_ref[...].astype(jnp.float32))
    t = a * dinv_ref[...] + b2_ref[...]
    m = jnp.max(t, axis=1, keepdims=True)
    e = jnp.exp(t - m)
    lse = jnp.log(jnp.sum(e, axis=1, keepdims=True)) + m
    o_ref[...] = t - lse


def _post(acc2, h2p, dinv, b2):
    d = h2p.shape[1]
    return pl.pallas_call(
        _post_body,
        grid=(_GRID,),
        in_specs=[
            pl.BlockSpec((NC, _BR, d), lambda i: (0, i, 0)),
            pl.BlockSpec((_BR, d), lambda i: (i, 0)),
            pl.BlockSpec((_BR, 1), lambda i: (i, 0)),
            pl.BlockSpec((1, d), lambda i: (0, 0)),
        ],
        out_specs=pl.BlockSpec((_BR, d), lambda i: (i, 0)),
        out_shape=jax.ShapeDtypeStruct((NACC, d), jnp.float32),
    )(acc2, h2p, dinv, b2)


def kernel(x, edge_index, W1, b1, W2, b2):
    n, d_in = x.shape
    e = edge_index.shape[1]
    d_h = W1.shape[1]
    d_out = W2.shape[1]

    xp = jnp.pad(x, ((0, NACC - n), (0, 0)))
    pad_e = EPAD - e
    src1 = jnp.concatenate([edge_index[0], jnp.zeros((pad_e,), jnp.int32)])
    dst1 = jnp.concatenate([edge_index[1], jnp.full((pad_e,), n, jnp.int32)])
    src2 = src1.reshape(TCH, K)
    dst2 = dst1.reshape(TCH, K)
    pad_ch = CH_PAD - TCH
    src2p = jnp.concatenate([src2, jnp.zeros((pad_ch, K), jnp.int32)])
    dst2p = jnp.concatenate([dst2, jnp.full((pad_ch, K), n, jnp.int32)])
    dst3 = dst2.reshape(NW, CPW, K)
    zeros_deg = jnp.zeros((NACC, DW), jnp.float32)

    deg_parts = _degree(dst3, zeros_deg)
    deg = deg_parts[0, :, 0] + deg_parts[1, :, 0]
    dinv = lax.rsqrt(deg + 1.0).reshape(NACC, 1)

    hb = _mm_scale(xp, W1, dinv)
    acc1 = _segment_sum2(src2p, dst2p, hb)
    h2p = _mid(acc1, hb, dinv, b1.reshape(1, d_h), W2)
    acc2 = _segment_sum(src2p, dst2p, h2p)
    out = _post(acc2, h2p, dinv, b2.reshape(1, d_out))
    return out[:n]


# speedup vs baseline: 1.2397x; 1.0064x over previous
import functools

import jax
import jax.numpy as jnp
from jax import lax
from jax.experimental import pallas as pl
from jax.experimental.pallas import tpu as pltpu
from jax.experimental.pallas import tpu_sc as plsc

NC = 2
NS = 16
NW = NC * NS
K = 128
CPW = 80
EPAD = NW * CPW * K
TCH = NW * CPW
CPP = TCH // NS
X0 = 120
X1 = CPP - X0
Y0 = 112
Y1 = CPP - Y0
CH_PAD = (NS - 1) * CPP + 2 * X0
NACC = 10112
RPTA = NACC // NS
DW = 16
DH = 64
ZR = 79

_mesh = functools.partial(
    plsc.VectorSubcoreMesh,
    core_axis_name="c", subcore_axis_name="s", num_cores=NC, num_subcores=NS,
)
_sc_params = pltpu.CompilerParams(use_tc_tiling_on_sc=False)


def _deg_body(dst_hbm, zeros_hbm, out_hbm, dst_v, ones_v, acc):
    c = lax.axis_index("c")
    s = lax.axis_index("s")
    wid = s * NC + c
    pltpu.sync_copy(dst_hbm.at[wid], dst_v)
    for i in range(K):
        ones_v[i] = jnp.ones((DW,), jnp.float32)
    pltpu.sync_copy(zeros_hbm.at[pl.ds(s * RPTA, RPTA)],
                    acc.at[pl.ds(s * RPTA, RPTA)])
    plsc.subcore_barrier()

    def chunk(j, _):
        pltpu.sync_copy(ones_v, acc.at[dst_v.at[j]], add=True)
        return ()

    lax.fori_loop(0, CPW, chunk, ())
    plsc.subcore_barrier()
    pltpu.sync_copy(acc.at[pl.ds(s * RPTA, RPTA)],
                    out_hbm.at[c, pl.ds(s * RPTA, RPTA)])


def _degree(dst3, zeros_deg):
    return pl.kernel(
        _deg_body,
        out_type=jax.ShapeDtypeStruct((NC, NACC, DW), jnp.float32),
        mesh=_mesh(),
        compiler_params=_sc_params,
        scratch_types=[
            pltpu.VMEM((CPW, K), jnp.int32),
            pltpu.VMEM((K, DW), jnp.float32),
            pltpu.VMEM_SHARED((NACC, DW), jnp.float32),
        ],
    )(dst3, zeros_deg)


def _fill_zeros(zbuf):
    for i in range(ZR):
        for k in range(DH // 16):
            zbuf[i, pl.ds(k * 16, 16)] = jnp.zeros((16,), jnp.float32)


def _zero_acc_local(zbuf, acc, s):
    for k in range(RPTA // ZR):
        pltpu.sync_copy(zbuf, acc.at[pl.ds(s * RPTA + k * ZR, ZR)])


NBUF = 2


def _seg_pass(h_hbm, src_v, dst_v, acc, bufs, sems, cnt):
    for b in range(NBUF):
        pltpu.async_copy(h_hbm.at[src_v.at[b]], bufs[b], sems[b])

    def step(i, _):
        g = i * NBUF
        for b in range(NBUF):
            j = g + b
            pltpu.make_async_copy(h_hbm.at[src_v.at[j]], bufs[b], sems[b]).wait()
            pltpu.sync_copy(bufs[b], acc.at[dst_v.at[j]], add=True)

            @pl.when(j + NBUF < cnt)
            def _():
                pltpu.async_copy(h_hbm.at[src_v.at[j + NBUF]], bufs[b], sems[b])
        return ()

    lax.fori_loop(0, cnt // NBUF, step, ())


def _seg_split(c, s, x0, x1):
    base = s * CPP + c * x0
    cnt = jnp.where(c == 0, x0, x1)
    return base, cnt


def _seg1_body(src_hbm, dst_hbm, hb_hbm, out_hbm,
               src_v, dst_v, buf0, buf1, zbuf, acc, sem0, sem1):
    c = lax.axis_index("c")
    s = lax.axis_index("s")
    base, cnt = _seg_split(c, s, X0, X1)
    pltpu.sync_copy(src_hbm.at[pl.ds(base, X0)], src_v)
    pltpu.sync_copy(dst_hbm.at[pl.ds(base, X0)], dst_v)
    for i in range(ZR):
        for k in range(128 // 32):
            zbuf[i, pl.ds(k * 32, 32)] = jnp.zeros((32,), jnp.bfloat16)
    for k in range(RPTA // ZR):
        pltpu.sync_copy(zbuf, acc.at[pl.ds(s * RPTA + k * ZR, ZR)])
    plsc.subcore_barrier()
    _seg_pass(hb_hbm, src_v, dst_v, acc, (buf0, buf1), (sem0, sem1), cnt)
    plsc.subcore_barrier()
    pltpu.sync_copy(acc.at[pl.ds(s * RPTA, RPTA)],
                    out_hbm.at[c, pl.ds(s * RPTA, RPTA)])


def _segment_sum2(src2p, dst2p, hb):
    d_h = hb.shape[1]
    return pl.kernel(
        _seg1_body,
        out_type=jax.ShapeDtypeStruct((NC, NACC, d_h), jnp.bfloat16),
        mesh=_mesh(),
        compiler_params=_sc_params,
        scratch_types=[
            pltpu.VMEM((X0, K), jnp.int32),
            pltpu.VMEM((X0, K), jnp.int32),
            pltpu.VMEM((K, d_h), jnp.bfloat16),
            pltpu.VMEM((K, d_h), jnp.bfloat16),
            pltpu.VMEM((ZR, d_h), jnp.bfloat16),
            pltpu.VMEM_SHARED((NACC, d_h), jnp.bfloat16),
            pltpu.SemaphoreType.DMA,
            pltpu.SemaphoreType.DMA,
        ],
    )(src2p, dst2p, hb)


def _seg2_body(src_hbm, dst_hbm, h_hbm, out_hbm,
               src_v, dst_v, buf0, buf1, zbuf, acc, sem0, sem1):
    c = lax.axis_index("c")
    s = lax.axis_index("s")
    base, cnt = _seg_split(c, s, Y0, Y1)
    pltpu.sync_copy(src_hbm.at[pl.ds(base, Y0)], src_v)
    pltpu.sync_copy(dst_hbm.at[pl.ds(base, Y0)], dst_v)
    for i in range(ZR):
        for k in range(DH // 32):
            zbuf[i, pl.ds(k * 32, 32)] = jnp.zeros((32,), jnp.bfloat16)
    _zero_acc_local(zbuf, acc, s)
    plsc.subcore_barrier()
    _seg_pass(h_hbm, src_v, dst_v, acc, (buf0, buf1), (sem0, sem1), cnt)
    plsc.subcore_barrier()
    pltpu.sync_copy(acc.at[pl.ds(s * RPTA, RPTA)],
                    out_hbm.at[c, pl.ds(s * RPTA, RPTA)])


def _segment_sum(src2p, dst2p, h):
    return pl.kernel(
        _seg2_body,
        out_type=jax.ShapeDtypeStruct((NC, NACC, DH), jnp.bfloat16),
        mesh=_mesh(),
        compiler_params=_sc_params,
        scratch_types=[
            pltpu.VMEM((Y0, K), jnp.int32),
            pltpu.VMEM((Y0, K), jnp.int32),
            pltpu.VMEM((K, DH), jnp.bfloat16),
            pltpu.VMEM((K, DH), jnp.bfloat16),
            pltpu.VMEM((ZR, DH), jnp.bfloat16),
            pltpu.VMEM_SHARED((NACC, DH), jnp.bfloat16),
            pltpu.SemaphoreType.DMA,
            pltpu.SemaphoreType.DMA,
        ],
    )(src2p, dst2p, h)


_BR = 632
_GRID = NACC // _BR


def _mm_scale_body(x_ref, w_ref, dinv_ref, o_ref):
    h = jnp.dot(x_ref[...], w_ref[...], preferred_element_type=jnp.float32)
    o_ref[...] = (h * dinv_ref[...]).astype(jnp.bfloat16)


def _mm_scale(xp, w, dinv):
    d_in, d_out = w.shape
    return pl.pallas_call(
        _mm_scale_body,
        grid=(_GRID,),
        in_specs=[
            pl.BlockSpec((_BR, d_in), lambda i: (i, 0)),
            pl.BlockSpec((d_in, d_out), lambda i: (0, 0)),
            pl.BlockSpec((_BR, 1), lambda i: (i, 0)),
        ],
        out_specs=pl.BlockSpec((_BR, d_out), lambda i: (i, 0)),
        out_shape=jax.ShapeDtypeStruct((NACC, d_out), jnp.bfloat16),
    )(xp, w, dinv)


def _mid_body(acc_ref, hb_ref, dinv_ref, b1_ref, w2_ref, o_ref):
    a = (acc_ref[0].astype(jnp.float32) + acc_ref[1].astype(jnp.float32)
         + hb_ref[...].astype(jnp.float32))
    z = a * dinv_ref[...] + b1_ref[...]
    z = jnp.maximum(z, 0.0)
    h2 = jnp.dot(z, w2_ref[...], preferred_element_type=jnp.float32)
    o_ref[...] = (h2 * dinv_ref[...]).astype(jnp.bfloat16)


def _mid(acc1, hb, dinv, b1, w2):
    d_in, d_out = w2.shape
    return pl.pallas_call(
        _mid_body,
        grid=(_GRID,),
        in_specs=[
            pl.BlockSpec((NC, _BR, d_in), lambda i: (0, i, 0)),
            pl.BlockSpec((_BR, d_in), lambda i: (i, 0)),
            pl.BlockSpec((_BR, 1), lambda i: (i, 0)),
            pl.BlockSpec((1, d_in), lambda i: (0, 0)),
            pl.BlockSpec((d_in, d_out), lambda i: (0, 0)),
        ],
        out_specs=pl.BlockSpec((_BR, d_out), lambda i: (i, 0)),
        out_shape=jax.ShapeDtypeStruct((NACC, d_out), jnp.bfloat16),
    )(acc1, hb, dinv, b1, w2)


def _post_body(acc_ref, hp_ref, dinv_ref, b2_ref, o_ref):
    a = (acc_ref[0].astype(jnp.float32) + acc_ref[1].astype(jnp.float32)
         + hp_ref[...].astype(jnp.float32))
    t = a * dinv_ref[...] + b2_ref[...]
    m = jnp.max(t, axis=1, keepdims=True)
    e = jnp.exp(t - m)
    lse = jnp.log(jnp.sum(e, axis=1, keepdims=True)) + m
    o_ref[...] = t - lse


def _post(acc2, h2p, dinv, b2):
    d = h2p.shape[1]
    return pl.pallas_call(
        _post_body,
        grid=(_GRID,),
        in_specs=[
            pl.BlockSpec((NC, _BR, d), lambda i: (0, i, 0)),
            pl.BlockSpec((_BR, d), lambda i: (i, 0)),
            pl.BlockSpec((_BR, 1), lambda i: (i, 0)),
            pl.BlockSpec((1, d), lambda i: (0, 0)),
        ],
        out_specs=pl.BlockSpec((_BR, d), lambda i: (i, 0)),
        out_shape=jax.ShapeDtypeStruct((NACC, d), jnp.float32),
    )(acc2, h2p, dinv, b2)


def kernel(x, edge_index, W1, b1, W2, b2):
    n, d_in = x.shape
    e = edge_index.shape[1]
    d_h = W1.shape[1]
    d_out = W2.shape[1]

    xp = jnp.pad(x, ((0, NACC - n), (0, 0)))
    pad_e = EPAD - e
    src1 = jnp.concatenate([edge_index[0], jnp.zeros((pad_e,), jnp.int32)])
    dst1 = jnp.concatenate([edge_index[1], jnp.full((pad_e,), n, jnp.int32)])
    src2 = src1.reshape(TCH, K)
    dst2 = dst1.reshape(TCH, K)
    pad_ch = CH_PAD - TCH
    src2p = jnp.concatenate([src2, jnp.zeros((pad_ch, K), jnp.int32)])
    dst2p = jnp.concatenate([dst2, jnp.full((pad_ch, K), n, jnp.int32)])
    dst3 = dst2.reshape(NW, CPW, K)
    zeros_deg = jnp.zeros((NACC, DW), jnp.float32)

    deg_parts = _degree(dst3, zeros_deg)
    deg = deg_parts[0, :, 0] + deg_parts[1, :, 0]
    dinv = lax.rsqrt(deg + 1.0).reshape(NACC, 1)

    hb = _mm_scale(xp, W1, dinv)
    acc1 = _segment_sum2(src2p, dst2p, hb)
    h2p = _mid(acc1, hb, dinv, b1.reshape(1, d_h), W2)
    acc2 = _segment_sum(src2p, dst2p, h2p)
    out = _post(acc2, h2p, dinv, b2.reshape(1, d_out))
    return out[:n]
